# Initial kernel scaffold; baseline (speedup 1.0000x reference)
import jax
import jax.numpy as jnp
from jax.experimental import pallas as pl


def kernel(x, edge_index, W1, b1, W2, b2, W3, b3, W4, b4):
    raise NotImplementedError("write your pallas kernel here")


# speedup vs baseline: 6.1038x; 6.1038x over previous
import functools

import jax
import jax.numpy as jnp
from jax import lax
from jax.experimental import pallas as pl
from jax.experimental.pallas import tpu as pltpu
from jax.experimental.pallas import tpu_sc as plsc

N = 10000
E = 320000
D = 128
NC = 2
NS = 16
HALF = N // NC
ACC_R = 5008
EPT = E // NS
CHUNK = 125
NCHUNK = EPT // CHUNK
RPT = ACC_R // NS

_sc_mesh = plsc.VectorSubcoreMesh(core_axis_name="c", subcore_axis_name="s")


@functools.partial(
    pl.kernel,
    out_type=jax.ShapeDtypeStruct((NC, NS, RPT, D), jnp.float32),
    mesh=_sc_mesh,
    scratch_types=[
        pltpu.VMEM((NCHUNK, CHUNK), jnp.int32),
        pltpu.VMEM((NCHUNK, CHUNK), jnp.int32),
        pltpu.VMEM((2, CHUNK, D), jnp.float32),
        pltpu.VMEM_SHARED((ACC_R, D), jnp.float32),
        pltpu.SemaphoreType.DMA,
        pltpu.SemaphoreType.DMA,
    ],
)
def _sc_segment_sum(x_hbm, src_hbm, dst_hbm, zeros_hbm, out_hbm,
                    src_v, dst_v, rows_v, acc, sem0, sem1):
    cid = lax.axis_index("c")
    sid = lax.axis_index("s")

    pltpu.sync_copy(zeros_hbm, acc.at[pl.ds(sid * RPT, RPT)])
    pltpu.sync_copy(src_hbm.at[sid], src_v)
    pltpu.sync_copy(dst_hbm.at[cid, sid], dst_v)
    plsc.subcore_barrier()

    pltpu.async_copy(x_hbm.at[src_v.at[0]], rows_v.at[0], sem0)

    def body(jj, _):
        j0 = 2 * jj
        j1 = j0 + 1
        pltpu.async_copy(x_hbm.at[src_v.at[j1]], rows_v.at[1], sem1)
        pltpu.make_async_copy(x_hbm.at[src_v.at[j0]], rows_v.at[0], sem0).wait()
        pltpu.sync_copy(rows_v.at[0], acc.at[dst_v.at[j0]], add=True)

        @pl.when(jj + 1 < NCHUNK // 2)
        def _prefetch():
            pltpu.async_copy(x_hbm.at[src_v.at[j0 + 2]], rows_v.at[0], sem0)

        pltpu.make_async_copy(x_hbm.at[src_v.at[j1]], rows_v.at[1], sem1).wait()
        pltpu.sync_copy(rows_v.at[1], acc.at[dst_v.at[j1]], add=True)
        return 0

    lax.fori_loop(0, NCHUNK // 2, body, 0)

    plsc.subcore_barrier()
    pltpu.sync_copy(acc.at[pl.ds(sid * RPT, RPT)], out_hbm.at[cid, sid])


def _mlp1_body(x_ref, agg_ref, w1_ref, b1_ref, w2_ref, b2_ref, o_ref):
    h = x_ref[...] + agg_ref[...]
    h = jnp.dot(h, w1_ref[...], preferred_element_type=jnp.float32) + b1_ref[...]
    h = jnp.maximum(h, 0.0)
    h = jnp.dot(h, w2_ref[...], preferred_element_type=jnp.float32) + b2_ref[...]
    o_ref[...] = jnp.maximum(h, 0.0)


def _mlp2_body(x_ref, agg_ref, w3_ref, b3_ref, w4_ref, b4_ref, o_ref):
    h = x_ref[...] + agg_ref[...]
    h = jnp.dot(h, w3_ref[...], preferred_element_type=jnp.float32) + b3_ref[...]
    h = jnp.maximum(h, 0.0)
    h = jnp.dot(h, w4_ref[...], preferred_element_type=jnp.float32) + b4_ref[...]
    m = jnp.max(h, axis=1, keepdims=True)
    s = jnp.sum(jnp.exp(h - m), axis=1, keepdims=True)
    o_ref[...] = h - m - jnp.log(s)


_mlp1 = pl.pallas_call(
    _mlp1_body, out_shape=jax.ShapeDtypeStruct((N, D), jnp.float32))
_mlp2 = pl.pallas_call(
    _mlp2_body, out_shape=jax.ShapeDtypeStruct((N, D), jnp.float32))


def _agg_from_out(out):
    return out.reshape(NC, ACC_R, D)[:, :HALF].reshape(N, D)


@jax.jit
def kernel(x, edge_index, W1, b1, W2, b2, W3, b3, W4, b4):
    src = edge_index[0].reshape(NS, NCHUNK, CHUNK)
    dst = edge_index[1]
    dst_loc = jnp.stack([
        jnp.where(dst < HALF, dst, HALF),
        jnp.where(dst >= HALF, dst - HALF, HALF),
    ]).reshape(NC, NS, NCHUNK, CHUNK)
    zeros = jnp.zeros((RPT, D), jnp.float32)
    agg1 = _agg_from_out(_sc_segment_sum(x, src, dst_loc, zeros))
    h1 = _mlp1(x, agg1, W1, b1.reshape(1, D), W2, b2.reshape(1, D))
    agg2 = _agg_from_out(_sc_segment_sum(h1, src, dst_loc, zeros))
    return _mlp2(h1, agg2, W3, b3.reshape(1, D), W4, b4.reshape(1, D))
